# baseline (speedup 1.0000x reference)
import functools

import jax
import jax.numpy as jnp
from jax import lax
from jax.experimental import pallas as pl
from jax.experimental.pallas import tpu as pltpu
from jax.experimental.pallas import tpu_sc as plsc

ATR_MULTIPLE = 2.0
_B, _D, _T, _C = 16384, 2500, 400, 4
_DP = 2560
_NC, _NS, _L = 2, 16, 16
_NW = _NC * _NS
_BPW = _B // _NW
_GCHUNK = 128
_NCHUNK = _BPW // _GCHUNK
_NVEC = _BPW // _L
_TPW = 13
_FBUF = 6


def _detile_body(dref, aref, dout, aout):
    t = pl.program_id(0)
    for c in range(_C):
        dout[pl.ds(c * _DP, _D)] = dref[0, c, :]
    r = t % 8
    aout[pl.ds(r * _DP, _D)] = aref[r, :]


def _detile(artr_t, data_t):
    return pl.pallas_call(
        _detile_body,
        grid=(_T,),
        in_specs=[
            pl.BlockSpec((1, _C, _D), lambda t: (t, 0, 0)),
            pl.BlockSpec((8, _D), lambda t: (t // 8, 0)),
        ],
        out_specs=[
            pl.BlockSpec((_C * _DP,), lambda t: (t,)),
            pl.BlockSpec((8 * _DP,), lambda t: (t // 8,)),
        ],
        out_shape=[jax.ShapeDtypeStruct((_T * _C * _DP,), jnp.float32),
                   jax.ShapeDtypeStruct((_T * _DP,), jnp.float32)],
    )(data_t, artr_t)


def _sc_body(date_hbm, time_hbm, pos_hbm, act_hbm, prev_hbm,
             artr_hbm, data_hbm, out_hbm,
             dv, tv, pv, av, sv, ia, idd, ga, gd, ov, sem):
    wid = lax.axis_index("s") * _NC + lax.axis_index("c")
    base = wid * _BPW
    pltpu.sync_copy(date_hbm.at[pl.ds(base, _BPW)], dv)
    pltpu.sync_copy(time_hbm.at[pl.ds(base, _BPW)], tv)
    pltpu.sync_copy(pos_hbm.at[pl.ds(base, _BPW)], pv)
    pltpu.sync_copy(act_hbm.at[pl.ds(base, _BPW)], av)
    pltpu.sync_copy(prev_hbm.at[pl.ds(base, _BPW)], sv)

    one_i = jnp.full((_L,), 1, jnp.int32)
    two_i = jnp.full((_L,), 2, jnp.int32)
    three_i = jnp.full((_L,), 3, jnp.int32)
    zero_f = jnp.zeros((_L,), jnp.float32)

    for i in range(_NVEC):
        r, c0 = divmod(i, _GCHUNK // _L)
        c0 *= _L
        d = dv[pl.ds(i * _L, _L)]
        t = tv[pl.ds(i * _L, _L)]
        ia[r, pl.ds(c0, _L)] = t * _DP + d
        p = pv[pl.ds(i * _L, _L)]
        a = av[pl.ds(i * _L, _L)]
        direction = jnp.sign(p + a)
        ch = jnp.where(p == zero_f, three_i,
                       jnp.where(direction > zero_f, one_i, two_i))
        idd[r, pl.ds(c0, _L)] = t * (_C * _DP) + ch * _DP + d

    cps = []
    for j in range(_NCHUNK):
        cps.append(pltpu.async_copy(artr_hbm.at[ia.at[j]], ga.at[j], sem))
        cps.append(pltpu.async_copy(data_hbm.at[idd.at[j]], gd.at[j], sem))
    for cp in cps:
        cp.wait()

    for i in range(_NVEC):
        r, c0 = divmod(i, _GCHUNK // _L)
        c0 *= _L
        p = pv[pl.ds(i * _L, _L)]
        a = av[pl.ds(i * _L, _L)]
        ps = sv[pl.ds(i * _L, _L)]
        artr_v = ga[r, pl.ds(c0, _L)] * ATR_MULTIPLE + 1.0
        rp = gd[r, pl.ds(c0, _L)]
        direction = jnp.sign(p + a)
        ps = jnp.where((ps != ps) & (direction != zero_f),
                       direction * jnp.float32(-jnp.inf), ps)
        stop = jnp.where(direction > zero_f,
                         jnp.maximum(ps, rp / artr_v),
                         jnp.minimum(ps, rp * artr_v))
        stop = jnp.where((stop != stop) | (direction == zero_f), ps, stop)
        ov[pl.ds(i * _L, _L)] = stop

    pltpu.sync_copy(ov, out_hbm.at[pl.ds(base, _BPW)])


@jax.jit
def _sc_kernel(date_idx, time_idx, position, action, prev_stop,
               artr_flat, data_flat):
    mesh = plsc.VectorSubcoreMesh(core_axis_name="c", subcore_axis_name="s",
                                  num_cores=_NC, num_subcores=_NS)
    return pl.kernel(
        _sc_body,
        out_type=jax.ShapeDtypeStruct((_B,), jnp.float32),
        mesh=mesh,
        scratch_types=[
            pltpu.VMEM((_BPW,), jnp.int32),
            pltpu.VMEM((_BPW,), jnp.int32),
            pltpu.VMEM((_BPW,), jnp.float32),
            pltpu.VMEM((_BPW,), jnp.float32),
            pltpu.VMEM((_BPW,), jnp.float32),
            pltpu.VMEM((_NCHUNK, _GCHUNK), jnp.int32),
            pltpu.VMEM((_NCHUNK, _GCHUNK), jnp.int32),
            pltpu.VMEM((_NCHUNK, _GCHUNK), jnp.float32),
            pltpu.VMEM((_NCHUNK, _GCHUNK), jnp.float32),
            pltpu.VMEM((_BPW,), jnp.float32),
            pltpu.SemaphoreType.DMA,
        ],
    )(date_idx, time_idx, position, action, prev_stop, artr_flat, data_flat)


def kernel(date_idx, time_idx, position, action, prev_stop, artr, data):
    data_flat, artr_flat = _detile(artr.T, data.transpose(1, 2, 0))
    return _sc_kernel(date_idx.astype(jnp.int32), time_idx.astype(jnp.int32),
                      position, action, prev_stop, artr_flat, data_flat)


# speedup vs baseline: 2.7987x; 2.7987x over previous
import functools

import jax
import jax.numpy as jnp
from jax import lax
from jax.experimental import pallas as pl
from jax.experimental.pallas import tpu as pltpu
from jax.experimental.pallas import tpu_sc as plsc

ATR_MULTIPLE = 2.0
_B, _D, _T, _C = 16384, 2500, 400, 4
_NC, _NS, _L = 2, 16, 16
_NW = _NC * _NS
_BPW = _B // _NW
_GCHUNK = 128
_NCHUNK = _BPW // _GCHUNK
_NVEC = _BPW // _L
_TPW = 13
_FBUF = 6


def _sc_body(date_hbm, time_hbm, pos_hbm, act_hbm, prev_hbm,
             tab_hbm, out_hbm,
             dv, tv, pv, av, sv, ia, idd, ga, gd, ov, sem):
    wid = lax.axis_index("s") * _NC + lax.axis_index("c")
    base = wid * _BPW
    pltpu.sync_copy(date_hbm.at[pl.ds(base, _BPW)], dv)
    pltpu.sync_copy(time_hbm.at[pl.ds(base, _BPW)], tv)
    pltpu.sync_copy(pos_hbm.at[pl.ds(base, _BPW)], pv)
    pltpu.sync_copy(act_hbm.at[pl.ds(base, _BPW)], av)
    pltpu.sync_copy(prev_hbm.at[pl.ds(base, _BPW)], sv)

    one_i = jnp.full((_L,), 1, jnp.int32)
    two_i = jnp.full((_L,), 2, jnp.int32)
    three_i = jnp.full((_L,), 3, jnp.int32)
    zero_f = jnp.zeros((_L,), jnp.float32)

    for i in range(_NVEC):
        r, c0 = divmod(i, _GCHUNK // _L)
        c0 *= _L
        d = dv[pl.ds(i * _L, _L)]
        t = tv[pl.ds(i * _L, _L)]
        ia[r, pl.ds(c0, _L)] = (t * 5 + 4) * _D + d
        p = pv[pl.ds(i * _L, _L)]
        a = av[pl.ds(i * _L, _L)]
        direction = jnp.sign(p + a)
        ch = jnp.where(p == zero_f, three_i,
                       jnp.where(direction > zero_f, one_i, two_i))
        idd[r, pl.ds(c0, _L)] = (t * 5 + ch) * _D + d

    cps = []
    for j in range(_NCHUNK):
        cps.append(pltpu.async_copy(tab_hbm.at[ia.at[j]], ga.at[j], sem))
        cps.append(pltpu.async_copy(tab_hbm.at[idd.at[j]], gd.at[j], sem))
    for cp in cps:
        cp.wait()

    for i in range(_NVEC):
        r, c0 = divmod(i, _GCHUNK // _L)
        c0 *= _L
        p = pv[pl.ds(i * _L, _L)]
        a = av[pl.ds(i * _L, _L)]
        ps = sv[pl.ds(i * _L, _L)]
        artr_v = ga[r, pl.ds(c0, _L)] * ATR_MULTIPLE + 1.0
        rp = gd[r, pl.ds(c0, _L)]
        direction = jnp.sign(p + a)
        ps = jnp.where((ps != ps) & (direction != zero_f),
                       direction * jnp.float32(-jnp.inf), ps)
        stop = jnp.where(direction > zero_f,
                         jnp.maximum(ps, rp / artr_v),
                         jnp.minimum(ps, rp * artr_v))
        stop = jnp.where((stop != stop) | (direction == zero_f), ps, stop)
        ov[pl.ds(i * _L, _L)] = stop

    pltpu.sync_copy(ov, out_hbm.at[pl.ds(base, _BPW)])


@jax.jit
def _sc_kernel(date_idx, time_idx, position, action, prev_stop, tab_flat):
    mesh = plsc.VectorSubcoreMesh(core_axis_name="c", subcore_axis_name="s",
                                  num_cores=_NC, num_subcores=_NS)
    return pl.kernel(
        _sc_body,
        out_type=jax.ShapeDtypeStruct((_B,), jnp.float32),
        mesh=mesh,
        scratch_types=[
            pltpu.VMEM((_BPW,), jnp.int32),
            pltpu.VMEM((_BPW,), jnp.int32),
            pltpu.VMEM((_BPW,), jnp.float32),
            pltpu.VMEM((_BPW,), jnp.float32),
            pltpu.VMEM((_BPW,), jnp.float32),
            pltpu.VMEM((_NCHUNK, _GCHUNK), jnp.int32),
            pltpu.VMEM((_NCHUNK, _GCHUNK), jnp.int32),
            pltpu.VMEM((_NCHUNK, _GCHUNK), jnp.float32),
            pltpu.VMEM((_NCHUNK, _GCHUNK), jnp.float32),
            pltpu.VMEM((_BPW,), jnp.float32),
            pltpu.SemaphoreType.DMA,
        ],
    )(date_idx, time_idx, position, action, prev_stop, tab_flat)


def kernel(date_idx, time_idx, position, action, prev_stop, artr, data):
    tab = jnp.concatenate(
        [data.transpose(1, 2, 0), artr.T[:, None, :]], axis=1)
    return _sc_kernel(date_idx.astype(jnp.int32), time_idx.astype(jnp.int32),
                      position, action, prev_stop, tab.reshape(-1))


# speedup vs baseline: 3.4016x; 1.2154x over previous
import functools

import jax
import jax.numpy as jnp
from jax import lax
from jax.experimental import pallas as pl
from jax.experimental.pallas import tpu as pltpu
from jax.experimental.pallas import tpu_sc as plsc

ATR_MULTIPLE = 2.0
_B, _D, _T, _C = 16384, 2500, 400, 4
_NC, _NS, _L = 2, 16, 16
_NW = _NC * _NS
_BPW = _B // _NW
_GCHUNK = 128
_NCHUNK = _BPW // _GCHUNK
_NVEC = _BPW // _L
_TPW = 13
_FBUF = 6


def _sc_body(date_hbm, time_hbm, pos_hbm, act_hbm, prev_hbm,
             artr_hbm, data_hbm, out_hbm,
             dv, tv, pv, av, sv, ia, idd, ga, gd, ov, sem):
    wid = lax.axis_index("s") * _NC + lax.axis_index("c")
    base = wid * _BPW
    pltpu.sync_copy(date_hbm.at[pl.ds(base, _BPW)], dv)
    pltpu.sync_copy(time_hbm.at[pl.ds(base, _BPW)], tv)
    pltpu.sync_copy(pos_hbm.at[pl.ds(base, _BPW)], pv)
    pltpu.sync_copy(act_hbm.at[pl.ds(base, _BPW)], av)
    pltpu.sync_copy(prev_hbm.at[pl.ds(base, _BPW)], sv)

    one_i = jnp.full((_L,), 1, jnp.int32)
    two_i = jnp.full((_L,), 2, jnp.int32)
    three_i = jnp.full((_L,), 3, jnp.int32)
    zero_f = jnp.zeros((_L,), jnp.float32)

    for i in range(_NVEC):
        r, c0 = divmod(i, _GCHUNK // _L)
        c0 *= _L
        d = dv[pl.ds(i * _L, _L)]
        t = tv[pl.ds(i * _L, _L)]
        ia[r, pl.ds(c0, _L)] = t * _D + d
        p = pv[pl.ds(i * _L, _L)]
        a = av[pl.ds(i * _L, _L)]
        direction = jnp.sign(p + a)
        ch = jnp.where(p == zero_f, three_i,
                       jnp.where(direction > zero_f, one_i, two_i))
        idd[r, pl.ds(c0, _L)] = t * (_C * _D) + ch * _D + d

    cps = []
    for j in range(_NCHUNK):
        cps.append(pltpu.async_copy(artr_hbm.at[ia.at[j]], ga.at[j], sem))
        cps.append(pltpu.async_copy(data_hbm.at[idd.at[j]], gd.at[j], sem))
    for cp in cps:
        cp.wait()

    for i in range(_NVEC):
        r, c0 = divmod(i, _GCHUNK // _L)
        c0 *= _L
        p = pv[pl.ds(i * _L, _L)]
        a = av[pl.ds(i * _L, _L)]
        ps = sv[pl.ds(i * _L, _L)]
        artr_v = ga[r, pl.ds(c0, _L)] * ATR_MULTIPLE + 1.0
        rp = gd[r, pl.ds(c0, _L)]
        direction = jnp.sign(p + a)
        ps = jnp.where((ps != ps) & (direction != zero_f),
                       direction * jnp.float32(-jnp.inf), ps)
        stop = jnp.where(direction > zero_f,
                         jnp.maximum(ps, rp / artr_v),
                         jnp.minimum(ps, rp * artr_v))
        stop = jnp.where((stop != stop) | (direction == zero_f), ps, stop)
        ov[pl.ds(i * _L, _L)] = stop

    pltpu.sync_copy(ov, out_hbm.at[pl.ds(base, _BPW)])


@jax.jit
def _sc_kernel(date_idx, time_idx, position, action, prev_stop,
               artr_flat, data_flat):
    mesh = plsc.VectorSubcoreMesh(core_axis_name="c", subcore_axis_name="s",
                                  num_cores=_NC, num_subcores=_NS)
    return pl.kernel(
        _sc_body,
        out_type=jax.ShapeDtypeStruct((_B,), jnp.float32),
        mesh=mesh,
        scratch_types=[
            pltpu.VMEM((_BPW,), jnp.int32),
            pltpu.VMEM((_BPW,), jnp.int32),
            pltpu.VMEM((_BPW,), jnp.float32),
            pltpu.VMEM((_BPW,), jnp.float32),
            pltpu.VMEM((_BPW,), jnp.float32),
            pltpu.VMEM((_NCHUNK, _GCHUNK), jnp.int32),
            pltpu.VMEM((_NCHUNK, _GCHUNK), jnp.int32),
            pltpu.VMEM((_NCHUNK, _GCHUNK), jnp.float32),
            pltpu.VMEM((_NCHUNK, _GCHUNK), jnp.float32),
            pltpu.VMEM((_BPW,), jnp.float32),
            pltpu.SemaphoreType.DMA,
        ],
    )(date_idx, time_idx, position, action, prev_stop, artr_flat, data_flat)


def kernel(date_idx, time_idx, position, action, prev_stop, artr, data):
    artr_flat = artr.T.reshape(-1)
    data_flat = data.transpose(1, 2, 0).reshape(-1)
    return _sc_kernel(date_idx.astype(jnp.int32), time_idx.astype(jnp.int32),
                      position, action, prev_stop, artr_flat, data_flat)
